# baseline (speedup 1.0000x reference)
import functools

import jax
import jax.numpy as jnp
from jax import lax
from jax.experimental import pallas as pl
from jax.experimental.pallas import tpu as pltpu
from jax.experimental.pallas import tpu_sc as plsc

_BATCH = 16384

_info = plsc.get_sparse_core_info()
_NC = _info.num_cores
_NS = _info.num_subcores
_NW = _NC * _NS
_B_PER_W = _BATCH // _NW
_CHUNK = 128
_NCHUNK = _B_PER_W // _CHUNK


def _sc_gather_kernel(betas_hbm, idx_hbm, out_hbm, idx_v, out_v, sem):
    wid = lax.axis_index("s") * _NC + lax.axis_index("c")
    base = wid * _B_PER_W
    pltpu.sync_copy(idx_hbm.at[pl.ds(base, _B_PER_W)], idx_v)
    pltpu.async_copy(betas_hbm.at[idx_v], out_v, sem).wait()
    pltpu.sync_copy(out_v, out_hbm.at[pl.ds(base, _B_PER_W)])


@jax.jit
def kernel(betas, t_int):
    idx = t_int.astype(jnp.int32)
    mesh = plsc.VectorSubcoreMesh(core_axis_name="c", subcore_axis_name="s")
    run = functools.partial(
        pl.kernel,
        mesh=mesh,
        out_type=jax.ShapeDtypeStruct((_BATCH,), jnp.float32),
        scratch_types=[
            pltpu.VMEM((_B_PER_W,), jnp.int32),
            pltpu.VMEM((_B_PER_W,), jnp.float32),
            pltpu.SemaphoreType.DMA,
        ],
    )(_sc_gather_kernel)
    return run(betas, idx)


# speedup vs baseline: 1.4458x; 1.4458x over previous
import functools

import jax
import jax.numpy as jnp
from jax import lax
from jax.experimental import pallas as pl
from jax.experimental.pallas import tpu as pltpu
from jax.experimental.pallas import tpu_sc as plsc

_BATCH = 16384
_TABLE = 1001

_info = plsc.get_sparse_core_info()
_NC = _info.num_cores
_NS = _info.num_subcores
_NW = _NC * _NS
_B_PER_W = _BATCH // _NW


def _sc_gather_kernel(betas_hbm, idx_hbm, out_hbm, table_s, idx_v, out_v, sem):
    sid = lax.axis_index("s")
    wid = sid * _NC + lax.axis_index("c")
    base = wid * _B_PER_W

    @pl.when(sid == 0)
    def _():
        pltpu.sync_copy(betas_hbm, table_s)

    pltpu.sync_copy(idx_hbm.at[pl.ds(base, _B_PER_W)], idx_v)
    plsc.subcore_barrier()
    pltpu.async_copy(table_s.at[idx_v], out_v, sem).wait()
    pltpu.sync_copy(out_v, out_hbm.at[pl.ds(base, _B_PER_W)])


@jax.jit
def kernel(betas, t_int):
    idx = t_int.astype(jnp.int32)
    mesh = plsc.VectorSubcoreMesh(core_axis_name="c", subcore_axis_name="s")
    run = functools.partial(
        pl.kernel,
        mesh=mesh,
        out_type=jax.ShapeDtypeStruct((_BATCH,), jnp.float32),
        scratch_types=[
            pltpu.VMEM_SHARED((_TABLE,), jnp.float32),
            pltpu.VMEM((_B_PER_W,), jnp.int32),
            pltpu.VMEM((_B_PER_W,), jnp.float32),
            pltpu.SemaphoreType.DMA,
        ],
    )(_sc_gather_kernel)
    return run(betas, idx)


# speedup vs baseline: 1.5383x; 1.0640x over previous
import functools

import jax
import jax.numpy as jnp
from jax import lax
from jax.experimental import pallas as pl
from jax.experimental.pallas import tpu as pltpu
from jax.experimental.pallas import tpu_sc as plsc

_BATCH = 16384
_TABLE = 1001

_info = plsc.get_sparse_core_info()
_NC = 1
_NS = _info.num_subcores
_NW = _NC * _NS
_B_PER_W = _BATCH // _NW


def _sc_gather_kernel(betas_hbm, idx_hbm, out_hbm, table_s, idx_v, out_v, sem):
    sid = lax.axis_index("s")
    wid = sid * _NC + lax.axis_index("c")
    base = wid * _B_PER_W

    @pl.when(sid == 0)
    def _():
        pltpu.sync_copy(betas_hbm, table_s)

    pltpu.sync_copy(idx_hbm.at[pl.ds(base, _B_PER_W)], idx_v)
    plsc.subcore_barrier()
    pltpu.async_copy(table_s.at[idx_v], out_v, sem).wait()
    pltpu.sync_copy(out_v, out_hbm.at[pl.ds(base, _B_PER_W)])


@jax.jit
def kernel(betas, t_int):
    idx = t_int.astype(jnp.int32)
    mesh = plsc.VectorSubcoreMesh(core_axis_name="c", subcore_axis_name="s", num_cores=1)
    run = functools.partial(
        pl.kernel,
        mesh=mesh,
        out_type=jax.ShapeDtypeStruct((_BATCH,), jnp.float32),
        scratch_types=[
            pltpu.VMEM_SHARED((_TABLE,), jnp.float32),
            pltpu.VMEM((_B_PER_W,), jnp.int32),
            pltpu.VMEM((_B_PER_W,), jnp.float32),
            pltpu.SemaphoreType.DMA,
        ],
    )(_sc_gather_kernel)
    return run(betas, idx)


# speedup vs baseline: 1.5802x; 1.0272x over previous
import functools

import jax
import jax.numpy as jnp
from jax import lax
from jax.experimental import pallas as pl
from jax.experimental.pallas import tpu as pltpu
from jax.experimental.pallas import tpu_sc as plsc

_BATCH = 16384
_TABLE = 1001

_info = plsc.get_sparse_core_info()
_NS = _info.num_subcores
_NW = _NS
_B_PER_W = _BATCH // _NW
_HALF = _B_PER_W // 2


def _sc_gather_kernel(betas_hbm, idx_hbm, out_hbm, table_s, idx_v, out_v,
                      sem_t, sem_g, sem_o):
    sid = lax.axis_index("s")
    base = sid * _B_PER_W

    @pl.when(sid == 0)
    def _():
        pltpu.async_copy(betas_hbm, table_s, sem_t)

    pltpu.sync_copy(idx_hbm.at[pl.ds(base, _B_PER_W)], idx_v)

    @pl.when(sid == 0)
    def _():
        pltpu.make_async_copy(betas_hbm, table_s, sem_t).wait()

    plsc.subcore_barrier()
    pltpu.async_copy(
        table_s.at[idx_v.at[pl.ds(0, _HALF)]], out_v.at[pl.ds(0, _HALF)], sem_g
    ).wait()
    o0 = pltpu.async_copy(
        out_v.at[pl.ds(0, _HALF)], out_hbm.at[pl.ds(base, _HALF)], sem_o
    )
    pltpu.async_copy(
        table_s.at[idx_v.at[pl.ds(_HALF, _HALF)]],
        out_v.at[pl.ds(_HALF, _HALF)],
        sem_g,
    ).wait()
    o1 = pltpu.async_copy(
        out_v.at[pl.ds(_HALF, _HALF)], out_hbm.at[pl.ds(base + _HALF, _HALF)], sem_o
    )
    o0.wait()
    o1.wait()


@jax.jit
def kernel(betas, t_int):
    idx = t_int.astype(jnp.int32)
    mesh = plsc.VectorSubcoreMesh(core_axis_name="c", subcore_axis_name="s", num_cores=1)
    run = functools.partial(
        pl.kernel,
        mesh=mesh,
        out_type=jax.ShapeDtypeStruct((_BATCH,), jnp.float32),
        scratch_types=[
            pltpu.VMEM_SHARED((_TABLE,), jnp.float32),
            pltpu.VMEM((_B_PER_W,), jnp.int32),
            pltpu.VMEM((_B_PER_W,), jnp.float32),
            pltpu.SemaphoreType.DMA,
            pltpu.SemaphoreType.DMA,
            pltpu.SemaphoreType.DMA,
        ],
    )(_sc_gather_kernel)
    return run(betas, idx)
